# baseline (speedup 1.0000x reference)
import functools

import jax
import jax.numpy as jnp
from jax import lax
from jax.experimental import pallas as pl
from jax.experimental.pallas import tpu as pltpu
from jax.experimental.pallas import tpu_sc as plsc

N = 10000
E = 160000
D = 128
K = 4
N_PAD = 10240
CH = 128
EP = 163840
ROWS = EP // CH
TILES = 32
RPT = ROWS // TILES
RPS = N_PAD // 16
CW = 16
CR = N_PAD * CW // D
CPS = CR // 16
BN = 400
GRID = N // BN


def _tab1_body(x_ref, w_ref, o_ref):
    xb = x_ref[...]
    for j in range(K):
        o_ref[j] = jnp.dot(xb, w_ref[j], preferred_element_type=jnp.float32)


def _tab2_body(p_ref, c_ref, x_ref, wb1_ref, b1_ref, w2_ref, t2_ref, h_ref):
    cn = c_ref[0] + c_ref[1]
    xb = x_ref[...]
    acc = p_ref[0] + p_ref[1]
    for k in range(K):
        z = jnp.dot(xb, wb1_ref[k], preferred_element_type=jnp.float32) + b1_ref[k]
        acc = acc + cn[:, k:k + 1] * z
    h = jnp.maximum(acc, 0.0)
    for j in range(K):
        t2_ref[j] = jnp.dot(h, w2_ref[j], preferred_element_type=jnp.float32)
    h_ref[...] = h


def _fin_body(p_ref, c_ref, h_ref, wb2_ref, b2_ref, o_ref):
    cn = c_ref[0] + c_ref[1]
    hb = h_ref[...]
    acc = p_ref[0] + p_ref[1]
    for k in range(K):
        z = jnp.dot(hb, wb2_ref[k], preferred_element_type=jnp.float32) + b2_ref[k]
        acc = acc + cn[:, k:k + 1] * z
    o_ref[...] = acc


def _tables_step1(x, w):
    return pl.pallas_call(
        _tab1_body,
        grid=(GRID,),
        in_specs=[
            pl.BlockSpec((BN, D), lambda i: (i, 0)),
            pl.BlockSpec((K, D, D), lambda i: (0, 0, 0)),
        ],
        out_specs=pl.BlockSpec((K, BN, D), lambda i: (0, i, 0)),
        out_shape=jax.ShapeDtypeStruct((K, N, D), jnp.float32),
    )(x, w)


def _tables_step2(p, cnt, x, wb1, b1, w2):
    return pl.pallas_call(
        _tab2_body,
        grid=(GRID,),
        in_specs=[
            pl.BlockSpec((2, BN, D), lambda i: (0, i, 0)),
            pl.BlockSpec((2, BN, CW), lambda i: (0, i, 0)),
            pl.BlockSpec((BN, D), lambda i: (i, 0)),
            pl.BlockSpec((K, D, D), lambda i: (0, 0, 0)),
            pl.BlockSpec((K, D), lambda i: (0, 0)),
            pl.BlockSpec((K, D, D), lambda i: (0, 0, 0)),
        ],
        out_specs=[
            pl.BlockSpec((K, BN, D), lambda i: (0, i, 0)),
            pl.BlockSpec((BN, D), lambda i: (i, 0)),
        ],
        out_shape=[
            jax.ShapeDtypeStruct((K, N, D), jnp.float32),
            jax.ShapeDtypeStruct((N, D), jnp.float32),
        ],
    )(p, cnt, x, wb1, b1, w2)


def _finish(p, cnt, h, wb2, b2):
    return pl.pallas_call(
        _fin_body,
        grid=(GRID,),
        in_specs=[
            pl.BlockSpec((2, BN, D), lambda i: (0, i, 0)),
            pl.BlockSpec((2, BN, CW), lambda i: (0, i, 0)),
            pl.BlockSpec((BN, D), lambda i: (i, 0)),
            pl.BlockSpec((K, D, D), lambda i: (0, 0, 0)),
            pl.BlockSpec((K, D), lambda i: (0, 0)),
        ],
        out_specs=pl.BlockSpec((BN, D), lambda i: (i, 0)),
        out_shape=jax.ShapeDtypeStruct((N, D), jnp.float32),
    )(p, cnt, h, wb2, b2)


def _sc_body(t_hbm, gidx_hbm, sidx_hbm, z_hbm, out_hbm,
             gidx_v, s0, s1, r0, r1, acc, g0, g1, e0, e1):
    cid = lax.axis_index("c")
    sid = lax.axis_index("s")
    tid = cid * 16 + sid
    base = tid * RPT
    pltpu.sync_copy(gidx_hbm.at[pl.ds(base, RPT)], gidx_v)
    pltpu.sync_copy(z_hbm, acc.at[pl.ds(sid * RPS, RPS)])
    plsc.subcore_barrier()

    pltpu.async_copy(t_hbm.at[gidx_v.at[0]], r0, g0)
    pltpu.async_copy(sidx_hbm.at[pl.ds(base, 1)], s0, e0)
    pltpu.async_copy(t_hbm.at[gidx_v.at[1]], r1, g1)
    pltpu.async_copy(sidx_hbm.at[pl.ds(base + 1, 1)], s1, e1)

    def step(j, carry):
        t0 = 2 * j
        t1 = t0 + 1
        pltpu.make_async_copy(t_hbm.at[gidx_v.at[t0]], r0, g0).wait()
        pltpu.make_async_copy(sidx_hbm.at[pl.ds(base + t0, 1)], s0, e0).wait()
        pltpu.sync_copy(r0, acc.at[s0.at[0]], add=True)

        @pl.when(j < RPT // 2 - 1)
        def _():
            pltpu.async_copy(t_hbm.at[gidx_v.at[t0 + 2]], r0, g0)
            pltpu.async_copy(sidx_hbm.at[pl.ds(base + t0 + 2, 1)], s0, e0)

        pltpu.make_async_copy(t_hbm.at[gidx_v.at[t1]], r1, g1).wait()
        pltpu.make_async_copy(sidx_hbm.at[pl.ds(base + t1, 1)], s1, e1).wait()
        pltpu.sync_copy(r1, acc.at[s1.at[0]], add=True)

        @pl.when(j < RPT // 2 - 1)
        def _():
            pltpu.async_copy(t_hbm.at[gidx_v.at[t1 + 2]], r1, g1)
            pltpu.async_copy(sidx_hbm.at[pl.ds(base + t1 + 2, 1)], s1, e1)

        return carry

    lax.fori_loop(0, RPT // 2, step, 0)
    plsc.subcore_barrier()

    def wstep(kk, carry):
        r = sid * RPS + kk * CH
        pltpu.sync_copy(acc.at[pl.ds(r, CH)], r0)
        pltpu.sync_copy(r0, out_hbm.at[pl.ds(cid * N_PAD + r, CH)])
        return carry

    lax.fori_loop(0, RPS // CH, wstep, 0)


@functools.cache
def _sc_gather_scatter():
    return pl.kernel(
        _sc_body,
        mesh=plsc.VectorSubcoreMesh(core_axis_name="c", subcore_axis_name="s"),
        out_type=jax.ShapeDtypeStruct((2 * N_PAD, D), jnp.float32),
        scratch_types=[
            pltpu.VMEM((RPT, CH), jnp.int32),
            pltpu.VMEM((1, CH), jnp.int32),
            pltpu.VMEM((1, CH), jnp.int32),
            pltpu.VMEM((CH, D), jnp.float32),
            pltpu.VMEM((CH, D), jnp.float32),
            pltpu.VMEM_SHARED((N_PAD, D), jnp.float32),
            pltpu.SemaphoreType.DMA,
            pltpu.SemaphoreType.DMA,
            pltpu.SemaphoreType.DMA,
            pltpu.SemaphoreType.DMA,
        ],
    )


def _cnt_body(id_hbm, ocv_hbm, sidx_hbm, z_hbm, out_hbm,
              ocv, s0, s1, c0, c1, cacc, g0, g1, e0, e1):
    cid = lax.axis_index("c")
    sid = lax.axis_index("s")
    tid = cid * 16 + sid
    base = tid * RPT
    pltpu.sync_copy(ocv_hbm.at[pl.ds(base, RPT)], ocv)
    pltpu.sync_copy(z_hbm.at[pl.ds(0, CPS)], cacc.at[pl.ds(sid * CPS, CPS)])
    plsc.subcore_barrier()

    pltpu.async_copy(id_hbm.at[ocv.at[0]], c0, g0)
    pltpu.async_copy(sidx_hbm.at[pl.ds(base, 1)], s0, e0)
    pltpu.async_copy(id_hbm.at[ocv.at[1]], c1, g1)
    pltpu.async_copy(sidx_hbm.at[pl.ds(base + 1, 1)], s1, e1)

    def step(j, carry):
        t0 = 2 * j
        t1 = t0 + 1
        pltpu.make_async_copy(id_hbm.at[ocv.at[t0]], c0, g0).wait()
        pltpu.make_async_copy(sidx_hbm.at[pl.ds(base + t0, 1)], s0, e0).wait()
        pltpu.sync_copy(c0, cacc.at[s0.at[0]], add=True)

        @pl.when(j < RPT // 2 - 1)
        def _():
            pltpu.async_copy(id_hbm.at[ocv.at[t0 + 2]], c0, g0)
            pltpu.async_copy(sidx_hbm.at[pl.ds(base + t0 + 2, 1)], s0, e0)

        pltpu.make_async_copy(id_hbm.at[ocv.at[t1]], c1, g1).wait()
        pltpu.make_async_copy(sidx_hbm.at[pl.ds(base + t1, 1)], s1, e1).wait()
        pltpu.sync_copy(c1, cacc.at[s1.at[0]], add=True)

        @pl.when(j < RPT // 2 - 1)
        def _():
            pltpu.async_copy(id_hbm.at[ocv.at[t1 + 2]], c1, g1)
            pltpu.async_copy(sidx_hbm.at[pl.ds(base + t1 + 2, 1)], s1, e1)

        return carry

    lax.fori_loop(0, RPT // 2, step, 0)
    plsc.subcore_barrier()

    r = sid * CPS
    pltpu.sync_copy(cacc.at[pl.ds(r, CPS)], c0.at[pl.ds(0, CPS)])
    pltpu.sync_copy(c0.at[pl.ds(0, CPS)], out_hbm.at[pl.ds(cid * CR + r, CPS)])


@functools.cache
def _sc_count():
    return pl.kernel(
        _cnt_body,
        mesh=plsc.VectorSubcoreMesh(core_axis_name="c", subcore_axis_name="s"),
        out_type=jax.ShapeDtypeStruct((2 * CR, D), jnp.float32),
        scratch_types=[
            pltpu.VMEM((RPT, CH), jnp.int32),
            pltpu.VMEM((1, CH), jnp.int32),
            pltpu.VMEM((1, CH), jnp.int32),
            pltpu.VMEM((CH, D), jnp.float32),
            pltpu.VMEM((CH, D), jnp.float32),
            pltpu.VMEM_SHARED((CR, D), jnp.float32),
            pltpu.SemaphoreType.DMA,
            pltpu.SemaphoreType.DMA,
            pltpu.SemaphoreType.DMA,
            pltpu.SemaphoreType.DMA,
        ],
    )


def kernel(x, edge_index, edge_attr, node_ids, W1, b1, W2, b2):
    src = edge_index[0]
    dst = edge_index[1]
    et = edge_attr

    pad = EP - E
    ar = jnp.arange(pad, dtype=jnp.int32)
    gidx = jnp.concatenate([et * N + src, ar % 128]).reshape(ROWS, CH)
    sidx = jnp.concatenate([dst, N + ar % (N_PAD - N)]).reshape(ROWS, CH)
    ohcol = jnp.concatenate([(dst % 8) * CW + et,
                             jnp.zeros((pad,), jnp.int32)]).reshape(ROWS, CH)
    sidx8 = jnp.concatenate([dst // 8, N // 8 + ar % (CR - N // 8)]).reshape(ROWS, CH)
    ident = jnp.eye(D, dtype=jnp.float32)
    zrows = jnp.zeros((RPS, D), jnp.float32)

    W1t = W1[:, :D, :]
    W1b = W1[:, D:, :]
    W2t = W2[:, :D, :]
    W2b = W2[:, D:, :]

    cnt = _sc_count()(ident, ohcol, sidx8, zrows)
    cnt = cnt.reshape(2, N_PAD, CW)
    t1 = _tables_step1(x, W1t)
    p1 = _sc_gather_scatter()(t1.reshape(K * N, D), gidx, sidx, zrows)
    t2, h = _tables_step2(p1.reshape(2, N_PAD, D), cnt, x, W1b, b1, W2t)
    p2 = _sc_gather_scatter()(t2.reshape(K * N, D), gidx, sidx, zrows)
    return _finish(p2.reshape(2, N_PAD, D), cnt, h, W2b, b2)


# speedup vs baseline: 1.3747x; 1.3747x over previous
import functools

import jax
import jax.numpy as jnp
from jax import lax
from jax.experimental import pallas as pl
from jax.experimental.pallas import tpu as pltpu
from jax.experimental.pallas import tpu_sc as plsc

N = 10000
E = 160000
D = 128
K = 4
N_PAD = 10240
CH = 128
EP = 163840
ROWS = EP // CH
TILES = 32
RPT = ROWS // TILES
RPS = N_PAD // 16
CW = 16
CR = N_PAD * CW // D
CPS = CR // 16
BN = 400
GRID = N // BN


def _tab1_body(x_ref, w_ref, o_ref):
    xb = x_ref[...]
    for j in range(K):
        o_ref[j] = jnp.dot(xb, w_ref[j], preferred_element_type=jnp.float32)


def _tab2_body(p_ref, c_ref, x_ref, wb1_ref, b1_ref, w2_ref, t2_ref, h_ref):
    cn = c_ref[0] + c_ref[1]
    xb = x_ref[...]
    acc = p_ref[0] + p_ref[1]
    for k in range(K):
        z = jnp.dot(xb, wb1_ref[k], preferred_element_type=jnp.float32) + b1_ref[k]
        acc = acc + cn[:, k:k + 1] * z
    h = jnp.maximum(acc, 0.0)
    for j in range(K):
        t2_ref[j] = jnp.dot(h, w2_ref[j], preferred_element_type=jnp.float32)
    h_ref[...] = h


def _fin_body(p_ref, c_ref, h_ref, wb2_ref, b2_ref, o_ref):
    cn = c_ref[0] + c_ref[1]
    hb = h_ref[...]
    acc = p_ref[0] + p_ref[1]
    for k in range(K):
        z = jnp.dot(hb, wb2_ref[k], preferred_element_type=jnp.float32) + b2_ref[k]
        acc = acc + cn[:, k:k + 1] * z
    o_ref[...] = acc


def _tables_step1(x, w):
    return pl.pallas_call(
        _tab1_body,
        grid=(GRID,),
        in_specs=[
            pl.BlockSpec((BN, D), lambda i: (i, 0)),
            pl.BlockSpec((K, D, D), lambda i: (0, 0, 0)),
        ],
        out_specs=pl.BlockSpec((K, BN, D), lambda i: (0, i, 0)),
        out_shape=jax.ShapeDtypeStruct((K, N, D), jnp.float32),
    )(x, w)


def _tables_step2(p, cnt, x, wb1, b1, w2):
    return pl.pallas_call(
        _tab2_body,
        grid=(GRID,),
        in_specs=[
            pl.BlockSpec((2, BN, D), lambda i: (0, i, 0)),
            pl.BlockSpec((2, BN, CW), lambda i: (0, i, 0)),
            pl.BlockSpec((BN, D), lambda i: (i, 0)),
            pl.BlockSpec((K, D, D), lambda i: (0, 0, 0)),
            pl.BlockSpec((K, D), lambda i: (0, 0)),
            pl.BlockSpec((K, D, D), lambda i: (0, 0, 0)),
        ],
        out_specs=[
            pl.BlockSpec((K, BN, D), lambda i: (0, i, 0)),
            pl.BlockSpec((BN, D), lambda i: (i, 0)),
        ],
        out_shape=[
            jax.ShapeDtypeStruct((K, N, D), jnp.float32),
            jax.ShapeDtypeStruct((N, D), jnp.float32),
        ],
    )(p, cnt, x, wb1, b1, w2)


def _finish(p, cnt, h, wb2, b2):
    return pl.pallas_call(
        _fin_body,
        grid=(GRID,),
        in_specs=[
            pl.BlockSpec((2, BN, D), lambda i: (0, i, 0)),
            pl.BlockSpec((2, BN, CW), lambda i: (0, i, 0)),
            pl.BlockSpec((BN, D), lambda i: (i, 0)),
            pl.BlockSpec((K, D, D), lambda i: (0, 0, 0)),
            pl.BlockSpec((K, D), lambda i: (0, 0)),
        ],
        out_specs=pl.BlockSpec((BN, D), lambda i: (i, 0)),
        out_shape=jax.ShapeDtypeStruct((N, D), jnp.float32),
    )(p, cnt, h, wb2, b2)


def _sc_body(t_hbm, gidx_hbm, sidx_hbm, z_hbm, out_hbm,
             gidx_v, s0, s1, r0, r1, acc, g0, g1, e0, e1):
    cid = lax.axis_index("c")
    sid = lax.axis_index("s")
    tid = cid * 16 + sid
    base = tid * RPT
    pltpu.sync_copy(gidx_hbm.at[pl.ds(base, RPT)], gidx_v)
    pltpu.sync_copy(z_hbm, acc.at[pl.ds(sid * RPS, RPS)])
    plsc.subcore_barrier()

    pltpu.async_copy(t_hbm.at[gidx_v.at[0]], r0, g0)
    pltpu.async_copy(sidx_hbm.at[pl.ds(base, 1)], s0, e0)
    pltpu.async_copy(t_hbm.at[gidx_v.at[1]], r1, g1)
    pltpu.async_copy(sidx_hbm.at[pl.ds(base + 1, 1)], s1, e1)

    def step(j, carry):
        t0 = 2 * j
        t1 = t0 + 1
        pltpu.make_async_copy(t_hbm.at[gidx_v.at[t0]], r0, g0).wait()
        pltpu.make_async_copy(sidx_hbm.at[pl.ds(base + t0, 1)], s0, e0).wait()
        pltpu.sync_copy(r0, acc.at[s0.at[0]], add=True)

        @pl.when(j < RPT // 2 - 1)
        def _():
            pltpu.async_copy(t_hbm.at[gidx_v.at[t0 + 2]], r0, g0)
            pltpu.async_copy(sidx_hbm.at[pl.ds(base + t0 + 2, 1)], s0, e0)

        pltpu.make_async_copy(t_hbm.at[gidx_v.at[t1]], r1, g1).wait()
        pltpu.make_async_copy(sidx_hbm.at[pl.ds(base + t1, 1)], s1, e1).wait()
        pltpu.sync_copy(r1, acc.at[s1.at[0]], add=True)

        @pl.when(j < RPT // 2 - 1)
        def _():
            pltpu.async_copy(t_hbm.at[gidx_v.at[t1 + 2]], r1, g1)
            pltpu.async_copy(sidx_hbm.at[pl.ds(base + t1 + 2, 1)], s1, e1)

        return carry

    lax.fori_loop(0, RPT // 2, step, 0)
    plsc.subcore_barrier()

    def wstep(kk, carry):
        r = sid * RPS + kk * CH
        pltpu.sync_copy(acc.at[pl.ds(r, CH)], r0)
        pltpu.sync_copy(r0, out_hbm.at[pl.ds(cid * N_PAD + r, CH)])
        return carry

    lax.fori_loop(0, RPS // CH, wstep, 0)


@functools.cache
def _sc_gather_scatter():
    return pl.kernel(
        _sc_body,
        mesh=plsc.VectorSubcoreMesh(core_axis_name="c", subcore_axis_name="s"),
        out_type=jax.ShapeDtypeStruct((2 * N_PAD, D), jnp.float32),
        scratch_types=[
            pltpu.VMEM((RPT, CH), jnp.int32),
            pltpu.VMEM((1, CH), jnp.int32),
            pltpu.VMEM((1, CH), jnp.int32),
            pltpu.VMEM((CH, D), jnp.float32),
            pltpu.VMEM((CH, D), jnp.float32),
            pltpu.VMEM_SHARED((N_PAD, D), jnp.float32),
            pltpu.SemaphoreType.DMA,
            pltpu.SemaphoreType.DMA,
            pltpu.SemaphoreType.DMA,
            pltpu.SemaphoreType.DMA,
        ],
    )


def _cnt_body(id_hbm, ocv_hbm, sidx_hbm, z_hbm, out_hbm,
              ocv, s0, s1, c0, c1, cacc, g0, g1, e0, e1):
    cid = lax.axis_index("c")
    sid = lax.axis_index("s")
    tid = cid * 16 + sid
    base = tid * RPT
    pltpu.sync_copy(ocv_hbm.at[pl.ds(base, RPT)], ocv)
    pltpu.sync_copy(z_hbm.at[pl.ds(0, CPS)], cacc.at[pl.ds(sid * CPS, CPS)])
    plsc.subcore_barrier()

    pltpu.async_copy(id_hbm.at[ocv.at[0]], c0, g0)
    pltpu.async_copy(sidx_hbm.at[pl.ds(base, 1)], s0, e0)
    pltpu.async_copy(id_hbm.at[ocv.at[1]], c1, g1)
    pltpu.async_copy(sidx_hbm.at[pl.ds(base + 1, 1)], s1, e1)

    def step(j, carry):
        t0 = 2 * j
        t1 = t0 + 1
        pltpu.make_async_copy(id_hbm.at[ocv.at[t0]], c0, g0).wait()
        pltpu.make_async_copy(sidx_hbm.at[pl.ds(base + t0, 1)], s0, e0).wait()
        pltpu.sync_copy(c0, cacc.at[s0.at[0]], add=True)

        @pl.when(j < RPT // 2 - 1)
        def _():
            pltpu.async_copy(id_hbm.at[ocv.at[t0 + 2]], c0, g0)
            pltpu.async_copy(sidx_hbm.at[pl.ds(base + t0 + 2, 1)], s0, e0)

        pltpu.make_async_copy(id_hbm.at[ocv.at[t1]], c1, g1).wait()
        pltpu.make_async_copy(sidx_hbm.at[pl.ds(base + t1, 1)], s1, e1).wait()
        pltpu.sync_copy(c1, cacc.at[s1.at[0]], add=True)

        @pl.when(j < RPT // 2 - 1)
        def _():
            pltpu.async_copy(id_hbm.at[ocv.at[t1 + 2]], c1, g1)
            pltpu.async_copy(sidx_hbm.at[pl.ds(base + t1 + 2, 1)], s1, e1)

        return carry

    lax.fori_loop(0, RPT // 2, step, 0)
    plsc.subcore_barrier()

    r = sid * CPS
    pltpu.sync_copy(cacc.at[pl.ds(r, CPS)], c0.at[pl.ds(0, CPS)])
    pltpu.sync_copy(c0.at[pl.ds(0, CPS)], out_hbm.at[pl.ds(cid * CR + r, CPS)])


@functools.cache
def _sc_count():
    return pl.kernel(
        _cnt_body,
        mesh=plsc.VectorSubcoreMesh(core_axis_name="c", subcore_axis_name="s"),
        out_type=jax.ShapeDtypeStruct((2 * CR, D), jnp.float32),
        scratch_types=[
            pltpu.VMEM((RPT, CH), jnp.int32),
            pltpu.VMEM((1, CH), jnp.int32),
            pltpu.VMEM((1, CH), jnp.int32),
            pltpu.VMEM((CH, D), jnp.float32),
            pltpu.VMEM((CH, D), jnp.float32),
            pltpu.VMEM_SHARED((CR, D), jnp.float32),
            pltpu.SemaphoreType.DMA,
            pltpu.SemaphoreType.DMA,
            pltpu.SemaphoreType.DMA,
            pltpu.SemaphoreType.DMA,
        ],
    )


def kernel(x, edge_index, edge_attr, node_ids, W1, b1, W2, b2):
    src = edge_index[0]
    dst = edge_index[1]
    et = edge_attr

    pad = EP - E
    ar = jnp.arange(pad, dtype=jnp.int32)
    gidx = jnp.concatenate([et * N + src, ar % 128]).reshape(ROWS, CH)
    sidx = jnp.concatenate([dst, N + ar % (N_PAD - N)]).reshape(ROWS, CH)
    tidv = (jnp.arange(EP, dtype=jnp.int32) // (EP // TILES)) * D
    ohcol = (jnp.concatenate([(dst % 8) * CW + et,
                              jnp.zeros((pad,), jnp.int32)]) + tidv).reshape(ROWS, CH)
    sidx8 = jnp.concatenate([dst // 8, N // 8 + ar % (CR - N // 8)]).reshape(ROWS, CH)
    ident = jnp.tile(jnp.eye(D, dtype=jnp.float32), (TILES, 1))
    zrows = jnp.zeros((RPS, D), jnp.float32)

    W1t = W1[:, :D, :]
    W1b = W1[:, D:, :]
    W2t = W2[:, :D, :]
    W2b = W2[:, D:, :]

    cnt = _sc_count()(ident, ohcol, sidx8, zrows)
    cnt = cnt.reshape(2, N_PAD, CW)
    t1 = _tables_step1(x, W1t)
    p1 = _sc_gather_scatter()(t1.reshape(K * N, D), gidx, sidx, zrows)
    t2, h = _tables_step2(p1.reshape(2, N_PAD, D), cnt, x, W1b, b1, W2t)
    p2 = _sc_gather_scatter()(t2.reshape(K * N, D), gidx, sidx, zrows)
    return _finish(p2.reshape(2, N_PAD, D), cnt, h, W2b, b2)


# speedup vs baseline: 2.1643x; 1.5744x over previous
import functools

import jax
import jax.numpy as jnp
from jax import lax
from jax.experimental import pallas as pl
from jax.experimental.pallas import tpu as pltpu
from jax.experimental.pallas import tpu_sc as plsc

N = 10000
E = 160000
D = 128
K = 4
N_PAD = 10240
CH = 128
EP = 163840
ROWS = EP // CH
TILES = 32
RPT = ROWS // TILES
RPS = N_PAD // 16
CW = 16
CR = N_PAD * CW // D
CPS = CR // 16
BN = 400
GRID = N // BN


def _tab1_body(x_ref, w_ref, o_ref):
    xb = x_ref[...]
    for j in range(K):
        o_ref[j] = jnp.dot(xb, w_ref[j], preferred_element_type=jnp.float32)


def _tab2_body(p_ref, c_ref, x_ref, wb1_ref, b1_ref, w2_ref, t2_ref, h_ref):
    cn = c_ref[0] + c_ref[1]
    xb = x_ref[...]
    acc = p_ref[0] + p_ref[1]
    for k in range(K):
        z = jnp.dot(xb, wb1_ref[k], preferred_element_type=jnp.float32) + b1_ref[k]
        acc = acc + cn[:, k:k + 1] * z
    h = jnp.maximum(acc, 0.0)
    for j in range(K):
        t2_ref[j] = jnp.dot(h, w2_ref[j], preferred_element_type=jnp.float32)
    h_ref[...] = h


def _fin_body(p_ref, c_ref, h_ref, wb2_ref, b2_ref, o_ref):
    cn = c_ref[0] + c_ref[1]
    hb = h_ref[...]
    acc = p_ref[0] + p_ref[1]
    for k in range(K):
        z = jnp.dot(hb, wb2_ref[k], preferred_element_type=jnp.float32) + b2_ref[k]
        acc = acc + cn[:, k:k + 1] * z
    o_ref[...] = acc


def _tables_step1(x, w):
    return pl.pallas_call(
        _tab1_body,
        grid=(GRID,),
        in_specs=[
            pl.BlockSpec((BN, D), lambda i: (i, 0)),
            pl.BlockSpec((K, D, D), lambda i: (0, 0, 0)),
        ],
        out_specs=pl.BlockSpec((K, BN, D), lambda i: (0, i, 0)),
        out_shape=jax.ShapeDtypeStruct((K, N, D), jnp.float32),
    )(x, w)


def _tables_step2(p, cnt, x, wb1, b1, w2):
    return pl.pallas_call(
        _tab2_body,
        grid=(GRID,),
        in_specs=[
            pl.BlockSpec((2, BN, D), lambda i: (0, i, 0)),
            pl.BlockSpec((2, BN, CW), lambda i: (0, i, 0)),
            pl.BlockSpec((BN, D), lambda i: (i, 0)),
            pl.BlockSpec((K, D, D), lambda i: (0, 0, 0)),
            pl.BlockSpec((K, D), lambda i: (0, 0)),
            pl.BlockSpec((K, D, D), lambda i: (0, 0, 0)),
        ],
        out_specs=[
            pl.BlockSpec((K, BN, D), lambda i: (0, i, 0)),
            pl.BlockSpec((BN, D), lambda i: (i, 0)),
        ],
        out_shape=[
            jax.ShapeDtypeStruct((K, N, D), jnp.float32),
            jax.ShapeDtypeStruct((N, D), jnp.float32),
        ],
    )(p, cnt, x, wb1, b1, w2)


def _finish(p, cnt, h, wb2, b2):
    return pl.pallas_call(
        _fin_body,
        grid=(GRID,),
        in_specs=[
            pl.BlockSpec((2, BN, D), lambda i: (0, i, 0)),
            pl.BlockSpec((2, BN, CW), lambda i: (0, i, 0)),
            pl.BlockSpec((BN, D), lambda i: (i, 0)),
            pl.BlockSpec((K, D, D), lambda i: (0, 0, 0)),
            pl.BlockSpec((K, D), lambda i: (0, 0)),
        ],
        out_specs=pl.BlockSpec((BN, D), lambda i: (i, 0)),
        out_shape=jax.ShapeDtypeStruct((N, D), jnp.float32),
    )(p, cnt, h, wb2, b2)


def _sc_body(t_hbm, gidx_hbm, sidx_hbm, z_hbm, out_hbm,
             gidx_v, s0, s1, r0, r1, acc, g0, g1, e0, e1):
    cid = lax.axis_index("c")
    sid = lax.axis_index("s")
    tid = cid * 16 + sid
    base = tid * RPT
    pltpu.sync_copy(gidx_hbm.at[pl.ds(base, RPT)], gidx_v)
    pltpu.sync_copy(z_hbm, acc.at[pl.ds(sid * RPS, RPS)])
    plsc.subcore_barrier()

    pltpu.async_copy(t_hbm.at[gidx_v.at[0]], r0, g0)
    pltpu.async_copy(sidx_hbm.at[pl.ds(base, 1)], s0, e0)
    pltpu.async_copy(t_hbm.at[gidx_v.at[1]], r1, g1)
    pltpu.async_copy(sidx_hbm.at[pl.ds(base + 1, 1)], s1, e1)

    def step(j, carry):
        t0 = 2 * j
        t1 = t0 + 1
        pltpu.make_async_copy(t_hbm.at[gidx_v.at[t0]], r0, g0).wait()
        pltpu.make_async_copy(sidx_hbm.at[pl.ds(base + t0, 1)], s0, e0).wait()
        pltpu.sync_copy(r0, acc.at[s0.at[0]], add=True)

        @pl.when(j < RPT // 2 - 1)
        def _():
            pltpu.async_copy(t_hbm.at[gidx_v.at[t0 + 2]], r0, g0)
            pltpu.async_copy(sidx_hbm.at[pl.ds(base + t0 + 2, 1)], s0, e0)

        pltpu.make_async_copy(t_hbm.at[gidx_v.at[t1]], r1, g1).wait()
        pltpu.make_async_copy(sidx_hbm.at[pl.ds(base + t1, 1)], s1, e1).wait()
        pltpu.sync_copy(r1, acc.at[s1.at[0]], add=True)

        @pl.when(j < RPT // 2 - 1)
        def _():
            pltpu.async_copy(t_hbm.at[gidx_v.at[t1 + 2]], r1, g1)
            pltpu.async_copy(sidx_hbm.at[pl.ds(base + t1 + 2, 1)], s1, e1)

        return carry

    lax.fori_loop(0, RPT // 2, step, 0)
    plsc.subcore_barrier()

    def wstep(kk, carry):
        r = sid * RPS + kk * CH
        pltpu.sync_copy(acc.at[pl.ds(r, CH)], r0)
        pltpu.sync_copy(r0, out_hbm.at[pl.ds(cid * N_PAD + r, CH)])
        return carry

    lax.fori_loop(0, RPS // CH, wstep, 0)


@functools.cache
def _sc_gather_scatter():
    return pl.kernel(
        _sc_body,
        mesh=plsc.VectorSubcoreMesh(core_axis_name="c", subcore_axis_name="s"),
        out_type=jax.ShapeDtypeStruct((2 * N_PAD, D), jnp.float32),
        scratch_types=[
            pltpu.VMEM((RPT, CH), jnp.int32),
            pltpu.VMEM((1, CH), jnp.int32),
            pltpu.VMEM((1, CH), jnp.int32),
            pltpu.VMEM((CH, D), jnp.float32),
            pltpu.VMEM((CH, D), jnp.float32),
            pltpu.VMEM_SHARED((N_PAD, D), jnp.float32),
            pltpu.SemaphoreType.DMA,
            pltpu.SemaphoreType.DMA,
            pltpu.SemaphoreType.DMA,
            pltpu.SemaphoreType.DMA,
        ],
    )


def _cnt_body(id_hbm, ocv_hbm, sidx_hbm, z_hbm, out_hbm,
              ocv, s0, s1, c0, c1, idv, cacc, g0, g1, e0, e1):
    cid = lax.axis_index("c")
    sid = lax.axis_index("s")
    tid = cid * 16 + sid
    base = tid * RPT
    pltpu.sync_copy(ocv_hbm.at[pl.ds(base, RPT)], ocv)
    @pl.when(sid == 0)
    def _():
        pltpu.sync_copy(id_hbm, idv)
    pltpu.sync_copy(z_hbm.at[pl.ds(0, CPS)], cacc.at[pl.ds(sid * CPS, CPS)])
    plsc.subcore_barrier()

    pltpu.async_copy(idv.at[ocv.at[0]], c0, g0)
    pltpu.async_copy(sidx_hbm.at[pl.ds(base, 1)], s0, e0)
    pltpu.async_copy(idv.at[ocv.at[1]], c1, g1)
    pltpu.async_copy(sidx_hbm.at[pl.ds(base + 1, 1)], s1, e1)

    def step(j, carry):
        t0 = 2 * j
        t1 = t0 + 1
        pltpu.make_async_copy(idv.at[ocv.at[t0]], c0, g0).wait()
        pltpu.make_async_copy(sidx_hbm.at[pl.ds(base + t0, 1)], s0, e0).wait()
        pltpu.sync_copy(c0, cacc.at[s0.at[0]], add=True)

        @pl.when(j < RPT // 2 - 1)
        def _():
            pltpu.async_copy(idv.at[ocv.at[t0 + 2]], c0, g0)
            pltpu.async_copy(sidx_hbm.at[pl.ds(base + t0 + 2, 1)], s0, e0)

        pltpu.make_async_copy(idv.at[ocv.at[t1]], c1, g1).wait()
        pltpu.make_async_copy(sidx_hbm.at[pl.ds(base + t1, 1)], s1, e1).wait()
        pltpu.sync_copy(c1, cacc.at[s1.at[0]], add=True)

        @pl.when(j < RPT // 2 - 1)
        def _():
            pltpu.async_copy(idv.at[ocv.at[t1 + 2]], c1, g1)
            pltpu.async_copy(sidx_hbm.at[pl.ds(base + t1 + 2, 1)], s1, e1)

        return carry

    lax.fori_loop(0, RPT // 2, step, 0)
    plsc.subcore_barrier()

    r = sid * CPS
    pltpu.sync_copy(cacc.at[pl.ds(r, CPS)], c0.at[pl.ds(0, CPS)])
    pltpu.sync_copy(c0.at[pl.ds(0, CPS)], out_hbm.at[pl.ds(cid * CR + r, CPS)])


@functools.cache
def _sc_count():
    return pl.kernel(
        _cnt_body,
        mesh=plsc.VectorSubcoreMesh(core_axis_name="c", subcore_axis_name="s"),
        out_type=jax.ShapeDtypeStruct((2 * CR, D), jnp.float32),
        scratch_types=[
            pltpu.VMEM((RPT, CH), jnp.int32),
            pltpu.VMEM((1, CH), jnp.int32),
            pltpu.VMEM((1, CH), jnp.int32),
            pltpu.VMEM((CH, D), jnp.float32),
            pltpu.VMEM((CH, D), jnp.float32),
            pltpu.VMEM_SHARED((D, D), jnp.float32),
            pltpu.VMEM_SHARED((CR, D), jnp.float32),
            pltpu.SemaphoreType.DMA,
            pltpu.SemaphoreType.DMA,
            pltpu.SemaphoreType.DMA,
            pltpu.SemaphoreType.DMA,
        ],
    )


def kernel(x, edge_index, edge_attr, node_ids, W1, b1, W2, b2):
    src = edge_index[0]
    dst = edge_index[1]
    et = edge_attr

    pad = EP - E
    ar = jnp.arange(pad, dtype=jnp.int32)
    gidx = jnp.concatenate([et * N + src, ar % 128]).reshape(ROWS, CH)
    sidx = jnp.concatenate([dst, N + ar % (N_PAD - N)]).reshape(ROWS, CH)
    ohcol = jnp.concatenate([(dst % 8) * CW + et,
                             jnp.zeros((pad,), jnp.int32)]).reshape(ROWS, CH)
    sidx8 = jnp.concatenate([dst // 8, N // 8 + ar % (CR - N // 8)]).reshape(ROWS, CH)
    ident = jnp.eye(D, dtype=jnp.float32)
    zrows = jnp.zeros((RPS, D), jnp.float32)

    W1t = W1[:, :D, :]
    W1b = W1[:, D:, :]
    W2t = W2[:, :D, :]
    W2b = W2[:, D:, :]

    cnt = _sc_count()(ident, ohcol, sidx8, zrows)
    cnt = cnt.reshape(2, N_PAD, CW)
    t1 = _tables_step1(x, W1t)
    p1 = _sc_gather_scatter()(t1.reshape(K * N, D), gidx, sidx, zrows)
    t2, h = _tables_step2(p1.reshape(2, N_PAD, D), cnt, x, W1b, b1, W2t)
    p2 = _sc_gather_scatter()(t2.reshape(K * N, D), gidx, sidx, zrows)
    return _finish(p2.reshape(2, N_PAD, D), cnt, h, W2b, b2)
